# baseline (speedup 1.0000x reference)
import functools

import jax
import jax.numpy as jnp
from jax import lax
from jax.experimental import pallas as pl
from jax.experimental.pallas import tpu as pltpu
from jax.experimental.pallas import tpu_sc as plsc

D_MODEL = 64
SCALE = 8.0
NC, NS, L = 2, 16, 16
NW = NC * NS
CH = 128


def _make_sc_lookup(n_ch: int, vocab: int):
    mesh = plsc.VectorSubcoreMesh(core_axis_name="c", subcore_axis_name="s")
    b_total = NW * n_ch * CH

    @functools.partial(
        pl.kernel,
        out_type=jax.ShapeDtypeStruct((b_total, D_MODEL), jnp.float32),
        mesh=mesh,
        scratch_types=[
            pltpu.VMEM((n_ch, CH), jnp.int32),
            pltpu.VMEM((CH, D_MODEL), jnp.float32),
            pltpu.SemaphoreType.DMA,
        ],
        compiler_params=pltpu.CompilerParams(use_tc_tiling_on_sc=False),
    )
    def k(x_hbm, table_hbm, out_hbm, idx_v, buf_v, gsem):
        wid = lax.axis_index("s") * NC + lax.axis_index("c")
        base = wid * (n_ch * CH)
        pltpu.sync_copy(x_hbm.at[wid], idx_v)

        def chunk(j, carry):
            pltpu.async_copy(table_hbm.at[idx_v.at[j]], buf_v, gsem).wait()

            def row(r, c2):
                for c in range(D_MODEL // L):
                    sl = pl.ds(c * L, L)
                    buf_v[r, sl] = buf_v[r, sl] * SCALE
                return c2

            lax.fori_loop(0, CH, row, 0)
            pltpu.sync_copy(buf_v, out_hbm.at[pl.ds(base + j * CH, CH)])
            return carry

        lax.fori_loop(0, n_ch, chunk, 0)

    return k


def kernel(x, table):
    b0, b1 = x.shape
    b_total = b0 * b1
    n_ch = b_total // (NW * CH)
    xr = x.reshape(NW, n_ch, CH).astype(jnp.int32)
    out = _make_sc_lookup(n_ch, table.shape[0])(xr, table)
    return out.reshape(b0, b1, D_MODEL)


# speedup vs baseline: 1.2079x; 1.2079x over previous
import functools

import jax
import jax.numpy as jnp
from jax import lax
from jax.experimental import pallas as pl
from jax.experimental.pallas import tpu as pltpu
from jax.experimental.pallas import tpu_sc as plsc

D_MODEL = 64
SCALE = 8.0
NC, NS, L = 2, 16, 16
NW = NC * NS
CH = 128
NBUF = 4


def _make_sc_lookup(n_ch: int):
    mesh = plsc.VectorSubcoreMesh(core_axis_name="c", subcore_axis_name="s")
    b_total = NW * n_ch * CH
    n_groups = n_ch // NBUF

    @functools.partial(
        pl.kernel,
        out_type=jax.ShapeDtypeStruct((b_total, D_MODEL), jnp.float32),
        mesh=mesh,
        scratch_types=[
            pltpu.VMEM((n_ch, CH), jnp.int32),
            [pltpu.VMEM((CH, D_MODEL), jnp.float32)] * NBUF,
            [pltpu.VMEM((CH, D_MODEL), jnp.float32)] * NBUF,
            [pltpu.SemaphoreType.DMA] * NBUF,
            [pltpu.SemaphoreType.DMA] * NBUF,
        ],
        compiler_params=pltpu.CompilerParams(use_tc_tiling_on_sc=False),
    )
    def k(x_hbm, table_hbm, out_hbm, idx_v, bufs, obufs, gsems, ssems):
        wid = lax.axis_index("s") * NC + lax.axis_index("c")
        base = wid * (n_ch * CH)
        pltpu.sync_copy(x_hbm.at[wid], idx_v)

        def g_start(j, b):
            pltpu.async_copy(table_hbm.at[idx_v.at[j]], bufs[b], gsems[b])

        def g_wait(j, b):
            pltpu.make_async_copy(
                table_hbm.at[idx_v.at[j]], bufs[b], gsems[b]
            ).wait()

        def s_start(j, b):
            pltpu.async_copy(
                obufs[b], out_hbm.at[pl.ds(base + j * CH, CH)], ssems[b]
            )

        def s_wait(j, b):
            pltpu.make_async_copy(
                obufs[b], out_hbm.at[pl.ds(base + j * CH, CH)], ssems[b]
            ).wait()

        def scale(b):
            buf, obuf = bufs[b], obufs[b]

            def srow(r, c2):
                for u in range(2):
                    for c in range(D_MODEL // L):
                        sl = pl.ds(c * L, L)
                        obuf[2 * r + u, sl] = buf[2 * r + u, sl] * SCALE
                return c2

            lax.fori_loop(0, CH // 2, srow, 0)

        for b in range(NBUF):
            g_start(b, b)

        for b in range(NBUF):
            g_wait(b, b)
            scale(b)
            g_start(b + NBUF, b)
            s_start(b, b)
            s_wait(b, b)

        def step(g, carry):
            for b in range(NBUF):
                j = g * NBUF + b
                g_wait(j, b)
                scale(b)
                g_start(j + NBUF, b)
                s_start(j, b)
                s_wait(j, b)
            return carry

        lax.fori_loop(1, n_groups - 1, step, 0)

        for b in range(NBUF):
            j = (n_groups - 1) * NBUF + b
            g_wait(j, b)
            scale(b)
            s_start(j, b)
            s_wait(j, b)

    return k


def kernel(x, table):
    b0, b1 = x.shape
    b_total = b0 * b1
    n_ch = b_total // (NW * CH)
    xr = x.reshape(NW, n_ch, CH).astype(jnp.int32)
    out = _make_sc_lookup(n_ch)(xr, table)
    return out.reshape(b0, b1, D_MODEL)
